# Initial kernel scaffold; baseline (speedup 1.0000x reference)
import jax
import jax.numpy as jnp
from jax.experimental import pallas as pl


def kernel(y_pred, y_true, weights, bin_edges):
    raise NotImplementedError("write your pallas kernel here")


# speedup vs baseline: 7.4114x; 7.4114x over previous
import functools

import jax
import jax.numpy as jnp
from jax import lax
from jax.experimental import pallas as pl
from jax.experimental.pallas import tpu as pltpu
from jax.experimental.pallas import tpu_sc as plsc

NC = 2
NS = 16
NW = NC * NS
L = 16

N = 4096 * 8192
PW = N // NW
CHUNK = 16384
NCH = PW // CHUNK
NV = CHUNK // L
UNROLL = 4

_mesh = plsc.VectorSubcoreMesh(core_axis_name="c", subcore_axis_name="s")


@functools.partial(
    pl.kernel,
    out_type=jax.ShapeDtypeStruct((NW, L), jnp.float32),
    mesh=_mesh,
    scratch_types=[
        pltpu.VMEM((48,), jnp.float32),
        pltpu.VMEM((CHUNK,), jnp.float32),
        pltpu.VMEM((CHUNK,), jnp.float32),
        pltpu.VMEM((CHUNK,), jnp.float32),
        pltpu.VMEM((CHUNK,), jnp.float32),
        pltpu.VMEM((L,), jnp.float32),
        pltpu.SemaphoreType.DMA,
        pltpu.SemaphoreType.DMA,
    ],
    compiler_params=pltpu.CompilerParams(needs_layout_passes=False),
)
def _sc_partial(yp_hbm, yt_hbm, tab_hbm, out_hbm,
                tab_v, ypA, ytA, ypB, ytB, accv, semA, semB):
    wid = lax.axis_index("s") * NC + lax.axis_index("c")
    base = wid * PW

    pltpu.sync_copy(tab_hbm, tab_v)
    invv = tab_v[pl.ds(32, L)]

    def start(buf_yp, buf_yt, sem, g):
        off = pl.multiple_of(base + g * CHUNK, CHUNK)
        pltpu.make_async_copy(yp_hbm.at[pl.ds(off, CHUNK)], buf_yp, sem).start()
        pltpu.make_async_copy(yt_hbm.at[pl.ds(off, CHUNK)], buf_yt, sem).start()

    def wait(buf_yp, buf_yt, sem):
        pltpu.make_async_copy(yp_hbm.at[pl.ds(0, CHUNK)], buf_yp, sem).wait()
        pltpu.make_async_copy(yt_hbm.at[pl.ds(0, CHUNK)], buf_yt, sem).wait()

    def compute(buf_yp, buf_yt, acc):
        def ibody(i, a):
            i0 = pl.multiple_of(i * (UNROLL * L), UNROLL * L)
            for u in range(UNROLL):
                off = i0 + u * L
                yv = buf_yt[pl.ds(off, L)]
                pv = buf_yp[pl.ds(off, L)]
                q = yv * invv
                c0 = q.astype(jnp.int32)
                e = c0.astype(jnp.float32)
                cnt = jnp.where(e < q, c0 + 1, c0)
                cnt = jnp.minimum(jnp.maximum(cnt, 0), 21)
                wv = plsc.load_gather(tab_v, [cnt])
                d = pv - yv
                a = a + wv * (d * d)
            return a
        return lax.fori_loop(0, NV // UNROLL, ibody, acc)

    start(ypA, ytA, semA, 0)

    def gbody(g2, acc):
        start(ypB, ytB, semB, 2 * g2 + 1)
        wait(ypA, ytA, semA)
        acc = compute(ypA, ytA, acc)

        @pl.when(g2 < NCH // 2 - 1)
        def _():
            start(ypA, ytA, semA, 2 * g2 + 2)

        wait(ypB, ytB, semB)
        acc = compute(ypB, ytB, acc)
        return acc

    acc = lax.fori_loop(0, NCH // 2, gbody, jnp.zeros((L,), jnp.float32))
    accv[...] = acc
    pltpu.sync_copy(accv, out_hbm.at[wid])


def kernel(y_pred, y_true, weights, bin_edges):
    yp = y_pred.reshape(-1)
    yt = y_true.reshape(-1)
    step = bin_edges[1] - bin_edges[0]
    inv_step = 1.0 / step
    tab = jnp.concatenate([
        weights[-1:], weights, weights[-1:],
        jnp.zeros((10,), jnp.float32),
        jnp.full((L,), inv_step, jnp.float32),
    ])
    partials = _sc_partial(yp, yt, tab)
    return jnp.sum(partials) / N
